# Initial kernel scaffold; baseline (speedup 1.0000x reference)
import jax
import jax.numpy as jnp
from jax.experimental import pallas as pl


def kernel(node_features, edge_index, W_in, b_in, W_u, b_u, W_o1, b_o1, W_o2, b_o2):
    raise NotImplementedError("write your pallas kernel here")


# speedup vs baseline: 5.3612x; 5.3612x over previous
import functools

import jax
import jax.numpy as jnp
from jax import lax
from jax.experimental import pallas as pl
from jax.experimental.pallas import tpu as pltpu
from jax.experimental.pallas import tpu_sc as plsc

N = 100000
E = 1600000
H = 64
L = 3

NC = 2
NS = 16
NW = NC * NS
EW = E // NW
CE = 10000
NCHUNK = EW // CE
VPC = CE // 16
NP = 5
R = 22400
AR = 22528
CAPB = EW // 128 + 1
CAP = CAPB * 128


def _worker_id():
  return lax.axis_index("s") * NC + lax.axis_index("c")


def _pre_body(src_hbm, dst_hbm, edgs, edgd, edgb, cnt_hbm,
              ebuf_s, ebuf_d, stages, cbuf):
  w = _worker_id()
  ebase = w * EW
  iot = lax.iota(jnp.int32, 16)

  def vec_body(v, carry):
    s = ebuf_s[pl.ds(v * 16, 16)]
    d = ebuf_d[pl.ds(v * 16, 16)]
    out = []
    for p in range(NP):
      cnt, blk = carry[2 * p], carry[2 * p + 1]
      st_s, st_d, st_b = stages[p]
      m = (d >= p * R) & (d < (p + 1) * R)
      mi = m.astype(jnp.int32)
      idx = plsc.cumsum(mi) + (cnt - 1)
      plsc.store_scatter(st_s, [idx], s, mask=m)
      plsc.store_scatter(st_d, [idx], d, mask=m)
      plsc.store_scatter(st_b, [idx], d - p * R, mask=m)
      cnt = cnt + jnp.sum(mi)
      do = cnt >= 128

      @pl.when(do)
      def _():
        off = ((w * NP + p) * CAPB + blk) * 128
        pltpu.sync_copy(st_s.at[pl.ds(0, 128)], edgs.at[pl.ds(off, 128)])
        pltpu.sync_copy(st_d.at[pl.ds(0, 128)], edgd.at[pl.ds(off, 128)])
        pltpu.sync_copy(st_b.at[pl.ds(0, 128)], edgb.at[pl.ds(off, 128)])
        for ref in (st_s, st_d, st_b):
          ref[pl.ds(0, 16)] = ref[pl.ds(128, 16)]

      cnt = jnp.where(do, cnt - 128, cnt)
      blk = jnp.where(do, blk + 1, blk)
      out += [cnt, blk]
    return tuple(out)

  carry = (jnp.int32(0),) * (2 * NP)
  for c in range(NCHUNK):
    pltpu.sync_copy(src_hbm.at[pl.ds(ebase + c * CE, CE)], ebuf_s)
    pltpu.sync_copy(dst_hbm.at[pl.ds(ebase + c * CE, CE)], ebuf_d)
    carry = lax.fori_loop(0, VPC, vec_body, carry)

  cntv = jnp.zeros((16,), jnp.int32)
  for p in range(NP):
    cnt, blk = carry[2 * p], carry[2 * p + 1]
    st_s, st_d, st_b = stages[p]
    for j in range(8):
      st_s[pl.ds(cnt + j * 16, 16)] = iot
      st_d[pl.ds(cnt + j * 16, 16)] = iot
      st_b[pl.ds(cnt + j * 16, 16)] = R + iot
    do = cnt > 0

    @pl.when(do)
    def _():
      off = ((w * NP + p) * CAPB + blk) * 128
      pltpu.sync_copy(st_s.at[pl.ds(0, 128)], edgs.at[pl.ds(off, 128)])
      pltpu.sync_copy(st_d.at[pl.ds(0, 128)], edgd.at[pl.ds(off, 128)])
      pltpu.sync_copy(st_b.at[pl.ds(0, 128)], edgb.at[pl.ds(off, 128)])

    blk = jnp.where(do, blk + 1, blk)
    cntv = jnp.where(iot == p, blk * 128, cntv)

  cbuf[pl.ds(0, 16)] = cntv
  pltpu.sync_copy(cbuf, cnt_hbm.at[w])


def _preprocess(src, dst):
  mesh = plsc.VectorSubcoreMesh(core_axis_name="c", subcore_axis_name="s")
  stages = [tuple(pltpu.VMEM((256,), jnp.int32) for _ in range(3))
            for _ in range(NP)]
  return pl.kernel(
      _pre_body,
      out_type=(
          jax.ShapeDtypeStruct((NW * NP * CAP,), jnp.int32),
          jax.ShapeDtypeStruct((NW * NP * CAP,), jnp.int32),
          jax.ShapeDtypeStruct((NW * NP * CAP,), jnp.int32),
          jax.ShapeDtypeStruct((NW, 16), jnp.int32),
      ),
      mesh=mesh,
      compiler_params=pltpu.CompilerParams(needs_layout_passes=False, use_tc_tiling_on_sc=False),
      scratch_types=[
          pltpu.VMEM((CE,), jnp.int32),
          pltpu.VMEM((CE,), jnp.int32),
          stages,
          pltpu.VMEM((16,), jnp.int32),
      ],
  )(src, dst)


def _layer_body(p_hbm, q_hbm, edgs, edgd, edgb, cnt_hbm, out_hbm,
                sidx, didx, bidx, pbuf, qbuf, ubuf, zbuf, cntv,
                acc, sem1, sem2):
  c = lax.axis_index("c")
  s = lax.axis_index("s")
  w = s * NC + c
  iot = lax.iota(jnp.int32, 16)
  zero16 = jnp.zeros((16,), jnp.float32)

  def zrow(r, _):
    for cc in range(4):
      zbuf[r, pl.ds(cc * 16, 16)] = zero16
    return 0

  lax.fori_loop(0, 128, zrow, 0)
  pltpu.sync_copy(cnt_hbm.at[w], cntv)

  for p in range(NP):
    for j in range(11):
      pltpu.sync_copy(zbuf, acc.at[pl.ds((s * 11 + j) * 128, 128)])
    plsc.subcore_barrier()

    cnt_p = jnp.sum(jnp.where(iot == p, cntv[pl.ds(0, 16)], 0))
    nblk = lax.shift_right_logical(cnt_p, 7)
    base_off = (w * NP + p) * CAP

    def blk_body(j, _):
      off = base_off + j * 128
      pltpu.sync_copy(edgs.at[pl.ds(off, 128)], sidx)
      pltpu.sync_copy(edgd.at[pl.ds(off, 128)], didx)
      pltpu.sync_copy(edgb.at[pl.ds(off, 128)], bidx)
      cp = pltpu.async_copy(p_hbm.at[sidx], pbuf, sem1)
      cq = pltpu.async_copy(q_hbm.at[didx], qbuf, sem2)
      cp.wait()
      cq.wait()

      def row_body(r, _):
        for cc in range(4):
          x = pbuf[r, pl.ds(cc * 16, 16)] + qbuf[r, pl.ds(cc * 16, 16)]
          e = jnp.exp(x + x)
          ubuf[r, pl.ds(cc * 16, 16)] = 1.0 - 2.0 / (e + 1.0)
        return 0

      lax.fori_loop(0, 128, row_body, 0)
      pltpu.sync_copy(ubuf, acc.at[bidx], add=True)
      return 0

    lax.fori_loop(0, nblk, blk_body, 0)
    plsc.subcore_barrier()

    for j in range(7):
      row = s * 1400 + j * 200
      pltpu.sync_copy(
          acc.at[pl.ds(row, 200)],
          out_hbm.at[pl.ds(c * (NP * R) + p * R + row, 200)])
    plsc.subcore_barrier()


def _edge_layer(P, Q, edgs, edgd, edgb, cnt):
  mesh = plsc.VectorSubcoreMesh(core_axis_name="c", subcore_axis_name="s")
  return pl.kernel(
      _layer_body,
      out_type=jax.ShapeDtypeStruct((NC * NP * R, H), jnp.float32),
      mesh=mesh,
      compiler_params=pltpu.CompilerParams(needs_layout_passes=False, use_tc_tiling_on_sc=False),
      scratch_types=[
          pltpu.VMEM((128,), jnp.int32),
          pltpu.VMEM((128,), jnp.int32),
          pltpu.VMEM((128,), jnp.int32),
          pltpu.VMEM((128, H), jnp.float32),
          pltpu.VMEM((128, H), jnp.float32),
          pltpu.VMEM((128, H), jnp.float32),
          pltpu.VMEM((128, H), jnp.float32),
          pltpu.VMEM((16,), jnp.int32),
          pltpu.VMEM_SHARED((AR, H), jnp.float32),
          pltpu.SemaphoreType.DMA,
          pltpu.SemaphoreType.DMA,
      ],
  )(P, Q, edgs, edgd, edgb, cnt)


_TB = 2048


def _first_body(x_ref, win_ref, bin_ref, wcat_ref, bcat_ref,
                h_ref, p_ref, q_ref):
  h = jnp.tanh(
      jnp.dot(x_ref[...], win_ref[...], preferred_element_type=jnp.float32)
      + bin_ref[...])
  h_ref[...] = h
  pq = jnp.dot(h, wcat_ref[...], preferred_element_type=jnp.float32)
  pq = pq + bcat_ref[...]
  p_ref[...] = pq[:, :H]
  q_ref[...] = pq[:, H:]


def _tc_first(x, W_in, b_in, Wcat, bcat):
  n = x.shape[0]
  grid = (pl.cdiv(n, _TB),)
  full = lambda shape: pl.BlockSpec(shape, lambda i: (0, 0))
  row = lambda width: pl.BlockSpec((_TB, width), lambda i: (i, 0))
  return pl.pallas_call(
      _first_body,
      grid=grid,
      in_specs=[row(4), full((4, H)), full((1, H)),
                full((H, 2 * H)), full((1, 2 * H))],
      out_specs=[row(H), row(H), row(H)],
      out_shape=[jax.ShapeDtypeStruct((n, H), jnp.float32)] * 3,
  )(x, W_in, b_in, Wcat, bcat)


def _mid_body(h_ref, o0_ref, o1_ref, wcat_ref, bcat_ref,
              h_out_ref, p_ref, q_ref):
  h = h_ref[...] + o0_ref[...] + o1_ref[...]
  h_out_ref[...] = h
  pq = jnp.dot(h, wcat_ref[...], preferred_element_type=jnp.float32)
  pq = pq + bcat_ref[...]
  p_ref[...] = pq[:, :H]
  q_ref[...] = pq[:, H:]


def _tc_mid(h, o0, o1, Wcat, bcat):
  n = h.shape[0]
  grid = (pl.cdiv(n, _TB),)
  full = lambda shape: pl.BlockSpec(shape, lambda i: (0, 0))
  row = lambda width: pl.BlockSpec((_TB, width), lambda i: (i, 0))
  return pl.pallas_call(
      _mid_body,
      grid=grid,
      in_specs=[row(H), row(H), row(H), full((H, 2 * H)), full((1, 2 * H))],
      out_specs=[row(H), row(H), row(H)],
      out_shape=[jax.ShapeDtypeStruct((n, H), jnp.float32)] * 3,
  )(h, o0, o1, Wcat, bcat)


def _final_body(h_ref, o0_ref, o1_ref, wo1_ref, bo1_ref, wo2_ref, bo2_ref,
                d_ref):
  h = h_ref[...] + o0_ref[...] + o1_ref[...]
  t = jnp.tanh(
      jnp.dot(h, wo1_ref[...], preferred_element_type=jnp.float32)
      + bo1_ref[...])
  d_ref[...] = (
      jnp.dot(t, wo2_ref[...], preferred_element_type=jnp.float32)
      + bo2_ref[...])


def _tc_final(h, o0, o1, W_o1, b_o1, W_o2, b_o2):
  n = h.shape[0]
  grid = (pl.cdiv(n, _TB),)
  full = lambda shape: pl.BlockSpec(shape, lambda i: (0, 0))
  row = lambda width: pl.BlockSpec((_TB, width), lambda i: (i, 0))
  return pl.pallas_call(
      _final_body,
      grid=grid,
      in_specs=[row(H), row(H), row(H), full((H, H)), full((1, H)),
                full((H, 8)), full((1, 8))],
      out_specs=row(8),
      out_shape=jax.ShapeDtypeStruct((n, 8), jnp.float32),
  )(h, o0, o1, W_o1, b_o1, W_o2, b_o2)


def kernel(node_features, edge_index, W_in, b_in, W_u, b_u, W_o1, b_o1,
           W_o2, b_o2):
  src = edge_index[0]
  dst = edge_index[1]
  edgs, edgd, edgb, cnt = _preprocess(src, dst)

  wcats = [jnp.concatenate([W_u[l, :H, :], W_u[l, H:, :]], axis=1)
           for l in range(L)]
  bcats = [jnp.concatenate([jnp.zeros((H,), jnp.float32), b_u[l]])
           .reshape(1, 2 * H) for l in range(L)]

  h, P, Q = _tc_first(node_features, W_in, b_in.reshape(1, H),
                      wcats[0], bcats[0])
  for l in range(L):
    out = _edge_layer(P, Q, edgs, edgd, edgb, cnt)
    o0 = out[:N]
    o1 = out[NP * R:NP * R + N]
    if l < L - 1:
      h, P, Q = _tc_mid(h, o0, o1, wcats[l + 1], bcats[l + 1])
    else:
      w_o2p = jnp.pad(W_o2, ((0, 0), (0, 8 - W_o2.shape[1])))
      b_o2p = jnp.pad(b_o2, (0, 8 - b_o2.shape[0])).reshape(1, 8)
      delta = _tc_final(h, o0, o1, W_o1, b_o1.reshape(1, H), w_o2p, b_o2p)
  return delta[:, :3]


# speedup vs baseline: 8.8641x; 1.6534x over previous
import jax
import jax.numpy as jnp
from jax import lax
from jax.experimental import pallas as pl
from jax.experimental.pallas import tpu as pltpu
from jax.experimental.pallas import tpu_sc as plsc

N = 100000
E = 1600000
H = 64
L = 3

NC = 2
NS = 16
NW = NC * NS
EW = E // NW
CE = 10000
NCHUNK = EW // CE
VPC = CE // 16
NP = 7
R = 16000
AR = 16384
CAPB = EW // 128 + 8
NQR = NP * R + 128

_SC_PARAMS = dict(
    compiler_params=pltpu.CompilerParams(
        needs_layout_passes=False, use_tc_tiling_on_sc=False),
)


def _worker_id():
  return lax.axis_index("s") * NC + lax.axis_index("c")


def _pre_body(src_hbm, dst_hbm, edgs, edgd, cnt_hbm,
              ebuf_s, ebuf_d, stages, cbuf):
  w = _worker_id()
  ebase = w * EW
  iot = lax.iota(jnp.int32, 16)

  def vec_body(v, carry):
    s = ebuf_s[pl.ds(v * 16, 16)]
    d = ebuf_d[pl.ds(v * 16, 16)]
    out = []
    for p in range(NP):
      cnt, blk = carry[2 * p], carry[2 * p + 1]
      st_s, st_d = stages[p]
      m = (d >= p * R) & (d < (p + 1) * R)
      mi = m.astype(jnp.int32)
      idx = plsc.cumsum(mi) + (cnt - 1)
      plsc.store_scatter(st_s, [idx], s, mask=m)
      plsc.store_scatter(st_d, [idx], d, mask=m)
      cnt = cnt + jnp.sum(mi)
      do = cnt >= 128

      @pl.when(do)
      def _():
        row = (w * NP + p) * CAPB + blk
        pltpu.sync_copy(st_s.at[pl.ds(0, 128)], edgs.at[row])
        pltpu.sync_copy(st_d.at[pl.ds(0, 128)], edgd.at[row])
        for ref in (st_s, st_d):
          ref[pl.ds(0, 16)] = ref[pl.ds(128, 16)]

      cnt = jnp.where(do, cnt - 128, cnt)
      blk = jnp.where(do, blk + 1, blk)
      out += [cnt, blk]
    return tuple(out)

  carry = (jnp.int32(0),) * (2 * NP)
  for c in range(NCHUNK):
    pltpu.sync_copy(src_hbm.at[pl.ds(ebase + c * CE, CE)], ebuf_s)
    pltpu.sync_copy(dst_hbm.at[pl.ds(ebase + c * CE, CE)], ebuf_d)
    carry = lax.fori_loop(0, VPC, vec_body, carry)

  cntv = jnp.zeros((16,), jnp.int32)
  for p in range(NP):
    cnt, blk = carry[2 * p], carry[2 * p + 1]
    st_s, st_d = stages[p]
    base = (w * NP + p) * CAPB
    for j in range(8):
      st_s[pl.ds(cnt + j * 16, 16)] = iot
      st_d[pl.ds(cnt + j * 16, 16)] = (p + 1) * R + iot
    do = cnt > 0

    @pl.when(do)
    def _():
      pltpu.sync_copy(st_s.at[pl.ds(0, 128)], edgs.at[base + blk])
      pltpu.sync_copy(st_d.at[pl.ds(0, 128)], edgd.at[base + blk])

    blk = jnp.where(do, blk + 1, blk)

    for j in range(8):
      st_s[pl.ds(j * 16, 16)] = iot
      st_d[pl.ds(j * 16, 16)] = (p + 1) * R + iot
    npad = (8 - (blk & 7)) & 7

    def padbody(i, _):
      pltpu.sync_copy(st_s.at[pl.ds(0, 128)], edgs.at[base + blk + i])
      pltpu.sync_copy(st_d.at[pl.ds(0, 128)], edgd.at[base + blk + i])
      return 0

    lax.fori_loop(0, npad, padbody, 0)
    blk = blk + npad
    cntv = jnp.where(iot == p, blk * 128, cntv)

  cbuf[pl.ds(0, 16)] = cntv
  pltpu.sync_copy(cbuf, cnt_hbm.at[w])


def _preprocess(src, dst):
  mesh = plsc.VectorSubcoreMesh(core_axis_name="c", subcore_axis_name="s")
  stages = [tuple(pltpu.VMEM((256,), jnp.int32) for _ in range(2))
            for _ in range(NP)]
  return pl.kernel(
      _pre_body,
      out_type=(
          jax.ShapeDtypeStruct((NW * NP * CAPB, 128), jnp.int32),
          jax.ShapeDtypeStruct((NW * NP * CAPB, 128), jnp.int32),
          jax.ShapeDtypeStruct((NW, 16), jnp.int32),
      ),
      mesh=mesh,
      scratch_types=[
          pltpu.VMEM((CE,), jnp.int32),
          pltpu.VMEM((CE,), jnp.int32),
          stages,
          pltpu.VMEM((16,), jnp.int32),
      ],
      **_SC_PARAMS,
  )(src, dst)


def _layer_body(p_hbm, q_hbm, edgs, edgd, cnt_hbm, out_hbm,
                sbuf, dbuf, bbuf, pbufs, qbufs, ubufs, zbuf, cntv,
                acc, sps, sqs, sus):
  c = lax.axis_index("c")
  s = lax.axis_index("s")
  w = s * NC + c
  iot = lax.iota(jnp.int32, 16)
  zero16 = jnp.zeros((16,), jnp.float32)

  def zrow(r, _):
    for cc in range(4):
      zbuf[r, pl.ds(cc * 16, 16)] = zero16
    return 0

  lax.fori_loop(0, 128, zrow, 0)
  pltpu.sync_copy(cnt_hbm.at[w], cntv)

  def compute_block(pb, qb, ub):
    def row_body(r, _):
      for cc in range(4):
        x = pb[r, pl.ds(cc * 16, 16)] + qb[r, pl.ds(cc * 16, 16)]
        e = jnp.exp(x + x)
        ub[r, pl.ds(cc * 16, 16)] = 1.0 - 2.0 / (e + 1.0)
      return 0

    lax.fori_loop(0, 128, row_body, 0)

  for p in range(NP):
    for j in range(8):
      pltpu.sync_copy(zbuf, acc.at[pl.ds((s * 8 + j) * 128, 128)])
    plsc.subcore_barrier()

    cnt_p = jnp.sum(jnp.where(iot == p, cntv[pl.ds(0, 16)], 0))
    nchunk = lax.shift_right_logical(cnt_p, 10)
    base_row = (w * NP + p) * CAPB

    def chunk_body(cix, _):
      rowbase = base_row + cix * 8
      pltpu.sync_copy(edgs.at[pl.ds(rowbase, 8)], sbuf)
      pltpu.sync_copy(edgd.at[pl.ds(rowbase, 8)], dbuf)

      def bb(r, _):
        for cc in range(8):
          bbuf[r, pl.ds(cc * 16, 16)] = dbuf[r, pl.ds(cc * 16, 16)] - p * R
        return 0

      lax.fori_loop(0, 8, bb, 0)

      gath = {}
      scat = {}
      for j in range(9):
        if j < 8:
          par = j & 1
          gath[j] = (
              pltpu.async_copy(p_hbm.at[sbuf.at[j]], pbufs[par], sps[par]),
              pltpu.async_copy(q_hbm.at[dbuf.at[j]], qbufs[par], sqs[par]))
        if j >= 1:
          k = j - 1
          kpar = k & 1
          for dsc in gath.pop(k):
            dsc.wait()
          if k >= 2:
            scat.pop(k - 2).wait()
          compute_block(pbufs[kpar], qbufs[kpar], ubufs[kpar])
          scat[k] = pltpu.async_copy(
              ubufs[kpar], acc.at[bbuf.at[k]], sus[kpar], add=True)
      scat.pop(6).wait()
      scat.pop(7).wait()
      return 0

    lax.fori_loop(0, nchunk, chunk_body, 0)
    plsc.subcore_barrier()

    for j in range(5):
      row = s * 1000 + j * 200
      pltpu.sync_copy(
          acc.at[pl.ds(row, 200)],
          out_hbm.at[pl.ds(c * (NP * R) + p * R + row, 200)])
    plsc.subcore_barrier()


def _edge_layer(P, Q, edgs, edgd, cnt):
  mesh = plsc.VectorSubcoreMesh(core_axis_name="c", subcore_axis_name="s")
  return pl.kernel(
      _layer_body,
      out_type=jax.ShapeDtypeStruct((NC * NP * R, H), jnp.float32),
      mesh=mesh,
      scratch_types=[
          pltpu.VMEM((8, 128), jnp.int32),
          pltpu.VMEM((8, 128), jnp.int32),
          pltpu.VMEM((8, 128), jnp.int32),
          [pltpu.VMEM((128, H), jnp.float32) for _ in range(2)],
          [pltpu.VMEM((128, H), jnp.float32) for _ in range(2)],
          [pltpu.VMEM((128, H), jnp.float32) for _ in range(2)],
          pltpu.VMEM((128, H), jnp.float32),
          pltpu.VMEM((16,), jnp.int32),
          pltpu.VMEM_SHARED((AR, H), jnp.float32),
          [pltpu.SemaphoreType.DMA for _ in range(2)],
          [pltpu.SemaphoreType.DMA for _ in range(2)],
          [pltpu.SemaphoreType.DMA for _ in range(2)],
      ],
      **_SC_PARAMS,
  )(P, Q, edgs, edgd, cnt)


_TB = 2048


def _first_body(x_ref, win_ref, bin_ref, wcat_ref, bcat_ref,
                h_ref, p_ref, q_ref):
  h = jnp.tanh(
      jnp.dot(x_ref[...], win_ref[...], preferred_element_type=jnp.float32)
      + bin_ref[...])
  h_ref[...] = h
  pq = jnp.dot(h, wcat_ref[...], preferred_element_type=jnp.float32)
  pq = pq + bcat_ref[...]
  p_ref[...] = pq[:, :H]
  q_ref[...] = pq[:, H:]


def _tc_first(x, W_in, b_in, Wcat, bcat):
  n = x.shape[0]
  grid = (pl.cdiv(n, _TB),)
  full = lambda shape: pl.BlockSpec(shape, lambda i: (0, 0))
  row = lambda width: pl.BlockSpec((_TB, width), lambda i: (i, 0))
  return pl.pallas_call(
      _first_body,
      grid=grid,
      in_specs=[row(4), full((4, H)), full((1, H)),
                full((H, 2 * H)), full((1, 2 * H))],
      out_specs=[row(H), row(H), row(H)],
      out_shape=[jax.ShapeDtypeStruct((n, H), jnp.float32),
                 jax.ShapeDtypeStruct((NQR, H), jnp.float32),
                 jax.ShapeDtypeStruct((NQR, H), jnp.float32)],
  )(x, W_in, b_in, Wcat, bcat)


def _mid_body(h_ref, o0_ref, o1_ref, wcat_ref, bcat_ref,
              h_out_ref, p_ref, q_ref):
  h = h_ref[...] + o0_ref[...] + o1_ref[...]
  h_out_ref[...] = h
  pq = jnp.dot(h, wcat_ref[...], preferred_element_type=jnp.float32)
  pq = pq + bcat_ref[...]
  p_ref[...] = pq[:, :H]
  q_ref[...] = pq[:, H:]


def _tc_mid(h, o0, o1, Wcat, bcat):
  n = h.shape[0]
  grid = (pl.cdiv(n, _TB),)
  full = lambda shape: pl.BlockSpec(shape, lambda i: (0, 0))
  row = lambda width: pl.BlockSpec((_TB, width), lambda i: (i, 0))
  return pl.pallas_call(
      _mid_body,
      grid=grid,
      in_specs=[row(H), row(H), row(H), full((H, 2 * H)), full((1, 2 * H))],
      out_specs=[row(H), row(H), row(H)],
      out_shape=[jax.ShapeDtypeStruct((n, H), jnp.float32),
                 jax.ShapeDtypeStruct((NQR, H), jnp.float32),
                 jax.ShapeDtypeStruct((NQR, H), jnp.float32)],
  )(h, o0, o1, Wcat, bcat)


def _final_body(h_ref, o0_ref, o1_ref, wo1_ref, bo1_ref, wo2_ref, bo2_ref,
                d_ref):
  h = h_ref[...] + o0_ref[...] + o1_ref[...]
  t = jnp.tanh(
      jnp.dot(h, wo1_ref[...], preferred_element_type=jnp.float32)
      + bo1_ref[...])
  d_ref[...] = (
      jnp.dot(t, wo2_ref[...], preferred_element_type=jnp.float32)
      + bo2_ref[...])


def _tc_final(h, o0, o1, W_o1, b_o1, W_o2, b_o2):
  n = h.shape[0]
  grid = (pl.cdiv(n, _TB),)
  full = lambda shape: pl.BlockSpec(shape, lambda i: (0, 0))
  row = lambda width: pl.BlockSpec((_TB, width), lambda i: (i, 0))
  return pl.pallas_call(
      _final_body,
      grid=grid,
      in_specs=[row(H), row(H), row(H), full((H, H)), full((1, H)),
                full((H, 8)), full((1, 8))],
      out_specs=row(8),
      out_shape=jax.ShapeDtypeStruct((n, 8), jnp.float32),
  )(h, o0, o1, W_o1, b_o1, W_o2, b_o2)


def kernel(node_features, edge_index, W_in, b_in, W_u, b_u, W_o1, b_o1,
           W_o2, b_o2):
  src = edge_index[0]
  dst = edge_index[1]
  edgs, edgd, cnt = _preprocess(src, dst)

  wcats = [jnp.concatenate([W_u[l, :H, :], W_u[l, H:, :]], axis=1)
           for l in range(L)]
  bcats = [jnp.concatenate([jnp.zeros((H,), jnp.float32), b_u[l]])
           .reshape(1, 2 * H) for l in range(L)]

  h, P, Q = _tc_first(node_features, W_in, b_in.reshape(1, H),
                      wcats[0], bcats[0])
  for l in range(L):
    out = _edge_layer(P, Q, edgs, edgd, cnt)
    o0 = out[:N]
    o1 = out[NP * R:NP * R + N]
    if l < L - 1:
      h, P, Q = _tc_mid(h, o0, o1, wcats[l + 1], bcats[l + 1])
    else:
      w_o2p = jnp.pad(W_o2, ((0, 0), (0, 8 - W_o2.shape[1])))
      b_o2p = jnp.pad(b_o2, (0, 8 - b_o2.shape[0])).reshape(1, 8)
      delta = _tc_final(h, o0, o1, W_o1, b_o1.reshape(1, H), w_o2p, b_o2p)
  return delta[:, :3]
